# baseline (speedup 1.0000x reference)
import functools

import jax
import jax.numpy as jnp
from jax import lax
from jax.experimental import pallas as pl
from jax.experimental.pallas import tpu as pltpu
from jax.experimental.pallas import tpu_sc as plsc

_NC = 2
_NS = 16
_NW = _NC * _NS
_L = 16


def _sc_mesh():
    return plsc.VectorSubcoreMesh(
        core_axis_name="c", subcore_axis_name="s",
        num_cores=_NC, num_subcores=_NS)


def _wid():
    return lax.axis_index("s") * _NC + lax.axis_index("c")


def _zero_ref(ref):
    zeros = jnp.zeros((_L,), jnp.float32)

    def body(i, carry):
        ref[pl.ds(i * _L, _L)] = zeros
        return carry

    lax.fori_loop(0, ref.shape[0] // _L, body, 0, unroll=4)


_ECH = 128


def _chunk_range(ei_hbm):
    ncw = ei_hbm.shape[1] // _ECH
    wid = _wid()
    c0 = wid * ncw // _NW
    c1 = (wid + 1) * ncw // _NW
    return c0, c1


def _deg_body(ei_hbm, out_hbm, ed_v, acc_v, sem):
    c0, c1 = _chunk_range(ei_hbm)
    cp = pltpu.async_copy(ei_hbm.at[:, pl.ds(c0 * _ECH, ed_v.shape[1])], ed_v, sem)
    _zero_ref(acc_v)
    cp.wait()
    ones = jnp.ones((_L,), jnp.float32)

    def body(ch, carry):
        for v in range(_ECH // _L):
            d = ed_v[1, pl.ds(ch * _ECH + v * _L, _L)]
            plsc.addupdate_scatter(acc_v, [d], ones)
        return carry

    lax.fori_loop(0, c1 - c0, body, 0)
    pltpu.sync_copy(acc_v, out_hbm.at[_wid()])


def _deg_call(edge_index, npad):
    e = edge_index.shape[1]
    cmax = e // _ECH // _NW + 1
    fn = pl.kernel(
        _deg_body,
        out_type=jax.ShapeDtypeStruct((_NW, npad), jnp.float32),
        mesh=_sc_mesh(),
        compiler_params=pltpu.CompilerParams(needs_layout_passes=False),
        scratch_types=[
            pltpu.VMEM((2, cmax * _ECH), jnp.int32),
            pltpu.VMEM((npad,), jnp.float32),
            pltpu.SemaphoreType.DMA,
        ],
    )
    return fn(edge_index)


def _msg_body(ei_hbm, g_hbm, out_hbm, ed_v, g_v, acc_v, sem):
    c0, c1 = _chunk_range(ei_hbm)
    cp1 = pltpu.async_copy(g_hbm, g_v, sem)
    cp2 = pltpu.async_copy(ei_hbm.at[:, pl.ds(c0 * _ECH, ed_v.shape[1])], ed_v, sem)
    _zero_ref(acc_v)
    cp1.wait()
    cp2.wait()

    def body(ch, carry):
        for v in range(_ECH // _L):
            s = ed_v[0, pl.ds(ch * _ECH + v * _L, _L)]
            d = ed_v[1, pl.ds(ch * _ECH + v * _L, _L)]
            vals = plsc.load_gather(g_v, [s])
            plsc.addupdate_scatter(acc_v, [d], vals)
        return carry

    lax.fori_loop(0, c1 - c0, body, 0)
    pltpu.sync_copy(acc_v, out_hbm.at[_wid()])


def _msg_call(edge_index, g, npad):
    e = edge_index.shape[1]
    cmax = e // _ECH // _NW + 1
    fn = pl.kernel(
        _msg_body,
        out_type=jax.ShapeDtypeStruct((_NW, npad), jnp.float32),
        mesh=_sc_mesh(),
        compiler_params=pltpu.CompilerParams(needs_layout_passes=False),
        scratch_types=[
            pltpu.VMEM((2, cmax * _ECH), jnp.int32),
            pltpu.VMEM((npad,), jnp.float32),
            pltpu.VMEM((npad,), jnp.float32),
            pltpu.SemaphoreType.DMA,
        ],
    )
    return fn(edge_index, g)


def _prep_body(x_ref, w_ref, degp_ref, g_ref, dis_ref):
    deg = jnp.sum(degp_ref[...], axis=0, keepdims=True) + 1.0
    dis = lax.rsqrt(deg)
    h = lax.dot_general(w_ref[...], x_ref[...], (((1,), (1,)), ((), ())),
                        preferred_element_type=jnp.float32)
    g_ref[...] = dis * h
    dis_ref[...] = dis


def _prep_call(x, w_row, degp, npad):
    n, d = x.shape
    bn = 1280
    shape = jax.ShapeDtypeStruct((1, npad), jnp.float32)
    blk = pl.BlockSpec((1, bn), lambda i: (0, i))
    return pl.pallas_call(
        _prep_body,
        grid=(npad // bn,),
        in_specs=[
            pl.BlockSpec((bn, d), lambda i: (i, 0)),
            pl.BlockSpec((1, d), lambda i: (0, 0)),
            pl.BlockSpec((_NW, bn), lambda i: (0, i)),
        ],
        out_specs=(blk, blk),
        out_shape=(shape, shape),
    )(x, w_row, degp)


def _fin_body(accp_ref, g_ref, dis_ref, b_ref, out_ref):
    n = out_ref.shape[1]
    tot = jnp.sum(accp_ref[:, :n], axis=0, keepdims=True)
    z = dis_ref[:, :n] * (tot + g_ref[:, :n]) + b_ref[0, 0]
    out_ref[...] = jnp.maximum(z, 0.0) + jnp.log1p(jnp.exp(-jnp.abs(z)))


def _fin_call(accp, g_row, dis_row, b, n):
    return pl.pallas_call(
        _fin_body,
        out_shape=jax.ShapeDtypeStruct((1, n), jnp.float32),
    )(accp, g_row, dis_row, b.reshape(1, 1))


@jax.jit
def kernel(x, edge_index, W, b):
    n, d = x.shape
    e = edge_index.shape[1]
    npad = -(-n // (_NS * _L)) * (_NS * _L)
    degp = _deg_call(edge_index, npad)
    g_row, dis_row = _prep_call(x, W.reshape(1, d), degp, npad)
    accp = _msg_call(edge_index, g_row.reshape(npad), npad)
    out_row = _fin_call(accp, g_row, dis_row, b, n)
    return out_row.reshape(n, 1)


# speedup vs baseline: 1.0594x; 1.0594x over previous
import functools

import jax
import jax.numpy as jnp
from jax import lax
from jax.experimental import pallas as pl
from jax.experimental.pallas import tpu as pltpu
from jax.experimental.pallas import tpu_sc as plsc

_NC = 2
_NS = 16
_NW = _NC * _NS
_L = 16


def _sc_mesh():
    return plsc.VectorSubcoreMesh(
        core_axis_name="c", subcore_axis_name="s",
        num_cores=_NC, num_subcores=_NS)


def _wid():
    return lax.axis_index("s") * _NC + lax.axis_index("c")


def _zero_ref(ref):
    zeros = jnp.zeros((_L,), jnp.float32)

    def body(i, carry):
        ref[pl.ds(i * _L, _L)] = zeros
        return carry

    lax.fori_loop(0, ref.shape[0] // _L, body, 0, unroll=4)


_ECH = 128


def _chunk_range(ei_hbm):
    ncw = ei_hbm.shape[1] // _ECH
    wid = _wid()
    c0 = wid * ncw // _NW
    c1 = (wid + 1) * ncw // _NW
    return c0, c1


def _deg_body(ei_hbm, out_hbm, ed_v, acc_v, sem):
    c0, c1 = _chunk_range(ei_hbm)
    cp = pltpu.async_copy(ei_hbm.at[:, pl.ds(c0 * _ECH, ed_v.shape[1])], ed_v, sem)
    _zero_ref(acc_v)
    cp.wait()
    ones = jnp.ones((_L,), jnp.float32)

    def body(ch, carry):
        for v in range(_ECH // _L):
            d = ed_v[1, pl.ds(ch * _ECH + v * _L, _L)]
            plsc.addupdate_scatter(acc_v, [d], ones)
        return carry

    lax.fori_loop(0, c1 - c0, body, 0)
    pltpu.sync_copy(acc_v, out_hbm.at[_wid()])


def _deg_call(edge_index, npad):
    e = edge_index.shape[1]
    cmax = e // _ECH // _NW + 1
    fn = pl.kernel(
        _deg_body,
        out_type=jax.ShapeDtypeStruct((_NW, npad), jnp.float32),
        mesh=_sc_mesh(),
        compiler_params=pltpu.CompilerParams(needs_layout_passes=False),
        scratch_types=[
            pltpu.VMEM((2, cmax * _ECH), jnp.int32),
            pltpu.VMEM((npad,), jnp.float32),
            pltpu.SemaphoreType.DMA,
        ],
    )
    return fn(edge_index)


def _msg_body(ei_hbm, g_hbm, out_hbm, ed_v, g_v, acc_v, sem):
    c0, c1 = _chunk_range(ei_hbm)
    cp1 = pltpu.async_copy(g_hbm, g_v, sem)
    cp2 = pltpu.async_copy(ei_hbm.at[:, pl.ds(c0 * _ECH, ed_v.shape[1])], ed_v, sem)
    _zero_ref(acc_v)
    cp1.wait()
    cp2.wait()

    def body(ch, carry):
        for v in range(_ECH // _L):
            s = ed_v[0, pl.ds(ch * _ECH + v * _L, _L)]
            d = ed_v[1, pl.ds(ch * _ECH + v * _L, _L)]
            vals = plsc.load_gather(g_v, [s])
            plsc.addupdate_scatter(acc_v, [d], vals)
        return carry

    lax.fori_loop(0, c1 - c0, body, 0)
    pltpu.sync_copy(acc_v, out_hbm.at[_wid()])


def _msg_call(edge_index, g, npad):
    e = edge_index.shape[1]
    cmax = e // _ECH // _NW + 1
    fn = pl.kernel(
        _msg_body,
        out_type=jax.ShapeDtypeStruct((_NW, npad), jnp.float32),
        mesh=_sc_mesh(),
        compiler_params=pltpu.CompilerParams(needs_layout_passes=False),
        scratch_types=[
            pltpu.VMEM((2, cmax * _ECH), jnp.int32),
            pltpu.VMEM((npad,), jnp.float32),
            pltpu.VMEM((npad,), jnp.float32),
            pltpu.SemaphoreType.DMA,
        ],
    )
    return fn(edge_index, g)


def _prep_body(x_ref, w_ref, degp_ref, g_ref, dis_ref):
    n = x_ref.shape[0]
    deg = jnp.sum(degp_ref[...], axis=0, keepdims=True) + 1.0
    dis = lax.rsqrt(deg)
    h = lax.dot_general(w_ref[...], x_ref[...], (((1,), (1,)), ((), ())),
                        preferred_element_type=jnp.float32)
    g_ref[...] = jnp.zeros_like(g_ref)
    g_ref[:, :n] = dis[:, :n] * h
    dis_ref[...] = dis


def _prep_call(x, w_row, degp, npad):
    shape = jax.ShapeDtypeStruct((1, npad), jnp.float32)
    return pl.pallas_call(
        _prep_body,
        out_shape=(shape, shape),
    )(x, w_row, degp)


def _fin_body(accp_ref, g_ref, dis_ref, b_ref, out_ref):
    n = out_ref.shape[1]
    tot = jnp.sum(accp_ref[:, :n], axis=0, keepdims=True)
    z = dis_ref[:, :n] * (tot + g_ref[:, :n]) + b_ref[0, 0]
    out_ref[...] = jnp.maximum(z, 0.0) + jnp.log1p(jnp.exp(-jnp.abs(z)))


def _fin_call(accp, g_row, dis_row, b, n):
    return pl.pallas_call(
        _fin_body,
        out_shape=jax.ShapeDtypeStruct((1, n), jnp.float32),
    )(accp, g_row, dis_row, b.reshape(1, 1))


@jax.jit
def kernel(x, edge_index, W, b):
    n, d = x.shape
    e = edge_index.shape[1]
    npad = -(-n // (_NS * _L)) * (_NS * _L)
    degp = _deg_call(edge_index, npad)
    g_row, dis_row = _prep_call(x, W.reshape(1, d), degp, npad)
    accp = _msg_call(edge_index, g_row.reshape(npad), npad)
    out_row = _fin_call(accp, g_row, dis_row, b, n)
    return out_row.reshape(n, 1)


# speedup vs baseline: 1.2430x; 1.1733x over previous
import functools

import jax
import jax.numpy as jnp
from jax import lax
from jax.experimental import pallas as pl
from jax.experimental.pallas import tpu as pltpu
from jax.experimental.pallas import tpu_sc as plsc

_NC = 2
_NS = 16
_NW = _NC * _NS
_L = 16


def _sc_mesh():
    return plsc.VectorSubcoreMesh(
        core_axis_name="c", subcore_axis_name="s",
        num_cores=_NC, num_subcores=_NS)


def _wid():
    return lax.axis_index("s") * _NC + lax.axis_index("c")


def _zero_ref(ref):
    zeros = jnp.zeros((_L,), jnp.float32)

    @plsc.parallel_loop(0, ref.shape[0] // _L, unroll=4)
    def _(i):
        ref[pl.ds(i * _L, _L)] = zeros


_ECH = 128


def _chunk_range(ei_hbm):
    ncw = ei_hbm.shape[1] // _ECH
    wid = _wid()
    c0 = wid * ncw // _NW
    c1 = (wid + 1) * ncw // _NW
    return c0, c1


def _deg_body(ei_hbm, out_hbm, ed_v, acc_v, sem):
    c0, c1 = _chunk_range(ei_hbm)
    cp = pltpu.async_copy(ei_hbm.at[:, pl.ds(c0 * _ECH, ed_v.shape[1])], ed_v, sem)
    _zero_ref(acc_v)
    cp.wait()
    ones = jnp.ones((_L,), jnp.float32)

    @plsc.parallel_loop(0, (c1 - c0) * (_ECH // _L), unroll=8)
    def _(i):
        d = ed_v[1, pl.ds(i * _L, _L)]
        plsc.addupdate_scatter(acc_v, [d], ones)

    pltpu.sync_copy(acc_v, out_hbm.at[_wid()])


def _deg_call(edge_index, npad):
    e = edge_index.shape[1]
    cmax = e // _ECH // _NW + 1
    fn = pl.kernel(
        _deg_body,
        out_type=jax.ShapeDtypeStruct((_NW, npad), jnp.float32),
        mesh=_sc_mesh(),
        compiler_params=pltpu.CompilerParams(needs_layout_passes=False),
        scratch_types=[
            pltpu.VMEM((2, cmax * _ECH), jnp.int32),
            pltpu.VMEM((npad,), jnp.float32),
            pltpu.SemaphoreType.DMA,
        ],
    )
    return fn(edge_index)


def _msg_body(ei_hbm, g_hbm, out_hbm, ed_v, g_v, acc_v, sem):
    c0, c1 = _chunk_range(ei_hbm)
    cp1 = pltpu.async_copy(g_hbm, g_v, sem)
    cp2 = pltpu.async_copy(ei_hbm.at[:, pl.ds(c0 * _ECH, ed_v.shape[1])], ed_v, sem)
    _zero_ref(acc_v)
    cp1.wait()
    cp2.wait()

    @plsc.parallel_loop(0, (c1 - c0) * (_ECH // _L), unroll=8)
    def _(i):
        s = ed_v[0, pl.ds(i * _L, _L)]
        d = ed_v[1, pl.ds(i * _L, _L)]
        vals = plsc.load_gather(g_v, [s])
        plsc.addupdate_scatter(acc_v, [d], vals)

    pltpu.sync_copy(acc_v, out_hbm.at[_wid()])


def _msg_call(edge_index, g, npad):
    e = edge_index.shape[1]
    cmax = e // _ECH // _NW + 1
    fn = pl.kernel(
        _msg_body,
        out_type=jax.ShapeDtypeStruct((_NW, npad), jnp.float32),
        mesh=_sc_mesh(),
        compiler_params=pltpu.CompilerParams(needs_layout_passes=False),
        scratch_types=[
            pltpu.VMEM((2, cmax * _ECH), jnp.int32),
            pltpu.VMEM((npad,), jnp.float32),
            pltpu.VMEM((npad,), jnp.float32),
            pltpu.SemaphoreType.DMA,
        ],
    )
    return fn(edge_index, g)


def _prep_body(x_ref, w_ref, degp_ref, g_ref, dis_ref):
    n = x_ref.shape[0]
    deg = jnp.sum(degp_ref[...], axis=0, keepdims=True) + 1.0
    dis = lax.rsqrt(deg)
    h = lax.dot_general(w_ref[...], x_ref[...], (((1,), (1,)), ((), ())),
                        preferred_element_type=jnp.float32)
    g_ref[...] = jnp.zeros_like(g_ref)
    g_ref[:, :n] = dis[:, :n] * h
    dis_ref[...] = dis


def _prep_call(x, w_row, degp, npad):
    shape = jax.ShapeDtypeStruct((1, npad), jnp.float32)
    return pl.pallas_call(
        _prep_body,
        out_shape=(shape, shape),
    )(x, w_row, degp)


def _fin_body(accp_ref, g_ref, dis_ref, b_ref, out_ref):
    n = out_ref.shape[1]
    tot = jnp.sum(accp_ref[:, :n], axis=0, keepdims=True)
    z = dis_ref[:, :n] * (tot + g_ref[:, :n]) + b_ref[0, 0]
    out_ref[...] = jnp.maximum(z, 0.0) + jnp.log1p(jnp.exp(-jnp.abs(z)))


def _fin_call(accp, g_row, dis_row, b, n):
    return pl.pallas_call(
        _fin_body,
        out_shape=jax.ShapeDtypeStruct((1, n), jnp.float32),
    )(accp, g_row, dis_row, b.reshape(1, 1))


@jax.jit
def kernel(x, edge_index, W, b):
    n, d = x.shape
    e = edge_index.shape[1]
    npad = -(-n // (_NS * _L)) * (_NS * _L)
    degp = _deg_call(edge_index, npad)
    g_row, dis_row = _prep_call(x, W.reshape(1, d), degp, npad)
    accp = _msg_call(edge_index, g_row.reshape(npad), npad)
    out_row = _fin_call(accp, g_row, dis_row, b, n)
    return out_row.reshape(n, 1)
